# Initial kernel scaffold; baseline (speedup 1.0000x reference)
import jax
import jax.numpy as jnp
from jax import lax
from jax.experimental import pallas as pl
from jax.experimental.pallas import tpu as pltpu
from jax.experimental.pallas import tpu_sc as plsc

B = 16384
D = 32
M = 4
NC = 2
NS = 16
NW = NC * NS
PW = B // NW
C = 256
NCH = PW // C
GROUPS = C // 16

_mesh = plsc.VectorSubcoreMesh(core_axis_name="c", subcore_axis_name="s")


def _body(uid_hbm, iid_hbm, taste_hbm, attn_hbm, item_hbm, ub_hbm, ib_hbm,
          out_hbm, uid_v, iid_v, ub_v, ib_v, out_v, taste_v, attn_v, item_v,
          sem):
    wid = lax.axis_index("s") * NC + lax.axis_index("c")
    base = wid * PW
    pltpu.sync_copy(uid_hbm.at[pl.ds(base, PW)], uid_v)
    pltpu.sync_copy(iid_hbm.at[pl.ds(base, PW)], iid_v)
    cp_ub = pltpu.async_copy(ub_hbm.at[uid_v], ub_v, sem)
    cp_ib = pltpu.async_copy(ib_hbm.at[iid_v], ib_v, sem)
    cp_ub.wait()
    cp_ib.wait()

    for c in range(NCH):
        cbase = c * C
        uidx = uid_v.at[pl.ds(cbase, C)]
        iidx = iid_v.at[pl.ds(cbase, C)]
        cp_t = pltpu.async_copy(taste_hbm.at[uidx], taste_v, sem)
        cp_a = pltpu.async_copy(attn_hbm.at[uidx], attn_v, sem)
        cp_i = pltpu.async_copy(item_hbm.at[iidx], item_v, sem)
        cp_t.wait()
        cp_a.wait()
        cp_i.wait()

        def group(g, carry, cbase=cbase):
            rows = jnp.full((16,), g * 16, jnp.int32) + lax.iota(jnp.int32, 16)
            zero = jnp.zeros((16,), jnp.float32)
            logits = [zero, zero, zero, zero]
            tdots = [zero, zero, zero, zero]
            for j in range(D):
                jc = jnp.full((16,), j, jnp.int32)
                iv = plsc.load_gather(item_v, [rows, jc])
                for m in range(M):
                    fc = jnp.full((16,), m * D + j, jnp.int32)
                    av = plsc.load_gather(attn_v, [rows, fc])
                    tv = plsc.load_gather(taste_v, [rows, fc])
                    logits[m] = logits[m] + av * iv
                    tdots[m] = tdots[m] + tv * iv
            mx = jnp.maximum(jnp.maximum(logits[0], logits[1]),
                             jnp.maximum(logits[2], logits[3]))
            e = [jnp.exp(l - mx) for l in logits]
            num = e[0] * tdots[0] + e[1] * tdots[1] + e[2] * tdots[2] + e[3] * tdots[3]
            den = e[0] + e[1] + e[2] + e[3]
            obase = cbase + g * 16
            res = num / den + ub_v[pl.ds(obase, 16)] + ib_v[pl.ds(obase, 16)]
            out_v[pl.ds(obase, 16)] = res
            return carry

        lax.fori_loop(0, GROUPS, group, None)

    pltpu.sync_copy(out_v, out_hbm.at[pl.ds(base, PW)])


_sc_call = pl.kernel(
    _body,
    out_type=jax.ShapeDtypeStruct((B,), jnp.float32),
    mesh=_mesh,
    scratch_types=[
        pltpu.VMEM((PW,), jnp.int32),
        pltpu.VMEM((PW,), jnp.int32),
        pltpu.VMEM((PW,), jnp.float32),
        pltpu.VMEM((PW,), jnp.float32),
        pltpu.VMEM((PW,), jnp.float32),
        pltpu.VMEM((C, M * D), jnp.float32),
        pltpu.VMEM((C, M * D), jnp.float32),
        pltpu.VMEM((C, D), jnp.float32),
        pltpu.SemaphoreType.DMA,
    ],
)


def kernel(user_ids, item_ids, taste_table, attention_table, item_table,
           user_biases, item_biases):
    uid = user_ids.astype(jnp.int32)
    iid = item_ids.astype(jnp.int32)
    ub = user_biases.reshape(-1)
    ib = item_biases.reshape(-1)
    return _sc_call(uid, iid, taste_table, attention_table, item_table, ub, ib)


# speedup vs baseline: 1.5736x; 1.5736x over previous
import jax
import jax.numpy as jnp
from jax import lax
from jax.experimental import pallas as pl
from jax.experimental.pallas import tpu as pltpu
from jax.experimental.pallas import tpu_sc as plsc

B = 16384
D = 32
M = 4
NC = 2
NS = 16
NW = NC * NS
PW = B // NW
NIT = 100000
C = 256
NCH = PW // C
GROUPS = C // 16

_mesh = plsc.VectorSubcoreMesh(core_axis_name="c", subcore_axis_name="s")


def _body(uid_hbm, iid_hbm, taste_hbm, attn_hbm, item_hbm, ub_hbm, ib_hbm,
          out_hbm, uid_v, iid_v, ub_v, ib_v, out_v, taste_v, attn_v, item_b,
          sem, isem):
    wid = lax.axis_index("s") * NC + lax.axis_index("c")
    base = wid * PW
    pltpu.sync_copy(uid_hbm.at[pl.ds(base, PW)], uid_v)
    pltpu.sync_copy(iid_hbm.at[pl.ds(base, PW)], iid_v)

    item_cps = []
    for j in range(D):
        item_cps.append(pltpu.async_copy(
            item_hbm.at[pl.ds(j * NIT, NIT)].at[iid_v],
            item_b.at[pl.ds(j * PW, PW)], isem))

    cp_ub = pltpu.async_copy(ub_hbm.at[uid_v], ub_v, sem)
    cp_ib = pltpu.async_copy(ib_hbm.at[iid_v], ib_v, sem)
    cp_ub.wait()
    cp_ib.wait()

    for c in range(NCH):
        cbase = c * C
        uidx = uid_v.at[pl.ds(cbase, C)]
        cp_t = pltpu.async_copy(taste_hbm.at[uidx], taste_v, sem)
        cp_a = pltpu.async_copy(attn_hbm.at[uidx], attn_v, sem)
        cp_t.wait()
        cp_a.wait()
        if c == 0:
            for cp in item_cps:
                cp.wait()

        def group(g, carry, cbase=cbase):
            lane = lax.iota(jnp.int32, 16)
            rows = jnp.full((16,), g * 16, jnp.int32) + lane
            ibase = jnp.full((16,), cbase + g * 16, jnp.int32) + lane
            zero = jnp.zeros((16,), jnp.float32)
            logits = [zero, zero, zero, zero]
            tdots = [zero, zero, zero, zero]
            for j in range(D):
                sk = (jnp.full((16,), j, jnp.int32) + lane) & (D - 1)
                iv = plsc.load_gather(item_b, [sk * PW + ibase])
                for m in range(M):
                    fc = sk + (m * D)
                    av = plsc.load_gather(attn_v, [rows, fc])
                    tv = plsc.load_gather(taste_v, [rows, fc])
                    logits[m] = logits[m] + av * iv
                    tdots[m] = tdots[m] + tv * iv
            mx = jnp.maximum(jnp.maximum(logits[0], logits[1]),
                             jnp.maximum(logits[2], logits[3]))
            e = [jnp.exp(l - mx) for l in logits]
            num = e[0] * tdots[0] + e[1] * tdots[1] + e[2] * tdots[2] + e[3] * tdots[3]
            den = e[0] + e[1] + e[2] + e[3]
            obase = cbase + g * 16
            res = num / den + ub_v[pl.ds(obase, 16)] + ib_v[pl.ds(obase, 16)]
            out_v[pl.ds(obase, 16)] = res
            return carry

        lax.fori_loop(0, GROUPS, group, None)

    pltpu.sync_copy(out_v, out_hbm.at[pl.ds(base, PW)])


_sc_call = pl.kernel(
    _body,
    out_type=jax.ShapeDtypeStruct((B,), jnp.float32),
    mesh=_mesh,
    scratch_types=[
        pltpu.VMEM((PW,), jnp.int32),
        pltpu.VMEM((PW,), jnp.int32),
        pltpu.VMEM((PW,), jnp.float32),
        pltpu.VMEM((PW,), jnp.float32),
        pltpu.VMEM((PW,), jnp.float32),
        pltpu.VMEM((C, M * D), jnp.float32),
        pltpu.VMEM((C, M * D), jnp.float32),
        pltpu.VMEM((D * PW,), jnp.float32),
        pltpu.SemaphoreType.DMA,
        pltpu.SemaphoreType.DMA,
    ],
    compiler_params=pltpu.CompilerParams(needs_layout_passes=False,
                                         use_tc_tiling_on_sc=False),
)


def kernel(user_ids, item_ids, taste_table, attention_table, item_table,
           user_biases, item_biases):
    uid = user_ids.astype(jnp.int32)
    iid = item_ids.astype(jnp.int32)
    ub = user_biases.reshape(-1)
    ib = item_biases.reshape(-1)
    item_feat = item_table.T.reshape(-1)
    return _sc_call(uid, iid, taste_table, attention_table, item_feat, ub, ib)


# speedup vs baseline: 1.6801x; 1.0677x over previous
import jax
import jax.numpy as jnp
from jax import lax
from jax.experimental import pallas as pl
from jax.experimental.pallas import tpu as pltpu
from jax.experimental.pallas import tpu_sc as plsc

B = 16384
D = 32
M = 4
NC = 2
NS = 16
NW = NC * NS
PW = B // NW
NIT = 100000
C = 128
NCH = PW // C
GROUPS = C // 16

_mesh = plsc.VectorSubcoreMesh(core_axis_name="c", subcore_axis_name="s")


def _body(uid_hbm, iid_hbm, taste_hbm, attn_hbm, item_hbm, ub_hbm, ib_hbm,
          out_hbm, uid_v, iid_v, ub_v, ib_v, out_v,
          taste0, taste1, attn0, attn1, item0, item1,
          bsem, sem0, sem1):
    wid = lax.axis_index("s") * NC + lax.axis_index("c")
    base = wid * PW
    pltpu.sync_copy(uid_hbm.at[pl.ds(base, PW)], uid_v)
    pltpu.sync_copy(iid_hbm.at[pl.ds(base, PW)], iid_v)

    taste_s = (taste0, taste1)
    attn_s = (attn0, attn1)
    item_s = (item0, item1)
    sems = (sem0, sem1)

    def fire(c):
        slot = c % 2
        cbase = c * C
        uidx = uid_v.at[pl.ds(cbase, C)]
        iidx = iid_v.at[pl.ds(cbase, C)]
        cps = [pltpu.async_copy(taste_hbm.at[uidx], taste_s[slot], sems[slot]),
               pltpu.async_copy(attn_hbm.at[uidx], attn_s[slot], sems[slot])]
        for j in range(D):
            cps.append(pltpu.async_copy(
                item_hbm.at[pl.ds(j * NIT, NIT)].at[iidx],
                item_s[slot].at[pl.ds(j * C, C)], sems[slot]))
        return cps

    cp_ub = pltpu.async_copy(ub_hbm.at[uid_v], ub_v, bsem)
    cp_ib = pltpu.async_copy(ib_hbm.at[iid_v], ib_v, bsem)
    pending = fire(0)

    for c in range(NCH):
        slot = c % 2
        cbase = c * C
        for cp in pending:
            cp.wait()
        if c + 1 < NCH:
            pending = fire(c + 1)
        if c == 0:
            cp_ub.wait()
            cp_ib.wait()
        taste_v, attn_v, item_b = taste_s[slot], attn_s[slot], item_s[slot]

        def group(g, carry, cbase=cbase, taste_v=taste_v, attn_v=attn_v,
                  item_b=item_b):
            lane = lax.iota(jnp.int32, 16)
            rows = jnp.full((16,), g * 16, jnp.int32) + lane
            zero = jnp.zeros((16,), jnp.float32)
            logits = [zero, zero, zero, zero]
            tdots = [zero, zero, zero, zero]
            for j in range(D):
                sk = (jnp.full((16,), j, jnp.int32) + lane) & (D - 1)
                iv = plsc.load_gather(item_b, [sk * C + rows])
                for m in range(M):
                    fc = sk + (m * D)
                    av = plsc.load_gather(attn_v, [rows, fc])
                    tv = plsc.load_gather(taste_v, [rows, fc])
                    logits[m] = logits[m] + av * iv
                    tdots[m] = tdots[m] + tv * iv
            mx = jnp.maximum(jnp.maximum(logits[0], logits[1]),
                             jnp.maximum(logits[2], logits[3]))
            e = [jnp.exp(l - mx) for l in logits]
            num = e[0] * tdots[0] + e[1] * tdots[1] + e[2] * tdots[2] + e[3] * tdots[3]
            den = e[0] + e[1] + e[2] + e[3]
            obase = cbase + g * 16
            res = num / den + ub_v[pl.ds(obase, 16)] + ib_v[pl.ds(obase, 16)]
            out_v[pl.ds(obase, 16)] = res
            return carry

        lax.fori_loop(0, GROUPS, group, None)

    pltpu.sync_copy(out_v, out_hbm.at[pl.ds(base, PW)])


_sc_call = pl.kernel(
    _body,
    out_type=jax.ShapeDtypeStruct((B,), jnp.float32),
    mesh=_mesh,
    scratch_types=[
        pltpu.VMEM((PW,), jnp.int32),
        pltpu.VMEM((PW,), jnp.int32),
        pltpu.VMEM((PW,), jnp.float32),
        pltpu.VMEM((PW,), jnp.float32),
        pltpu.VMEM((PW,), jnp.float32),
        pltpu.VMEM((C, M * D), jnp.float32),
        pltpu.VMEM((C, M * D), jnp.float32),
        pltpu.VMEM((C, M * D), jnp.float32),
        pltpu.VMEM((C, M * D), jnp.float32),
        pltpu.VMEM((D * C,), jnp.float32),
        pltpu.VMEM((D * C,), jnp.float32),
        pltpu.SemaphoreType.DMA,
        pltpu.SemaphoreType.DMA,
        pltpu.SemaphoreType.DMA,
    ],
    compiler_params=pltpu.CompilerParams(needs_layout_passes=False,
                                         use_tc_tiling_on_sc=False),
)


def kernel(user_ids, item_ids, taste_table, attention_table, item_table,
           user_biases, item_biases):
    uid = user_ids.astype(jnp.int32)
    iid = item_ids.astype(jnp.int32)
    ub = user_biases.reshape(-1)
    ib = item_biases.reshape(-1)
    item_feat = item_table.T.reshape(-1)
    return _sc_call(uid, iid, taste_table, attention_table, item_feat, ub, ib)


# speedup vs baseline: 1.8203x; 1.0834x over previous
import jax
import jax.numpy as jnp
from jax import lax
from jax.experimental import pallas as pl
from jax.experimental.pallas import tpu as pltpu
from jax.experimental.pallas import tpu_sc as plsc

B = 16384
D = 32
M = 4
NC = 2
NS = 16
NW = NC * NS
PW = B // NW
NIT = 100000
C = 64
NCH = PW // C
GROUPS = C // 16

_mesh = plsc.VectorSubcoreMesh(core_axis_name="c", subcore_axis_name="s")


def _body(uid_hbm, iid_hbm, taste_hbm, attn_hbm, item_hbm, ub_hbm, ib_hbm,
          out_hbm, uid_v, iid_v, ub_v, ib_v, out_v,
          taste0, taste1, attn0, attn1, item0, item1,
          bsem, sem0, sem1):
    wid = lax.axis_index("s") * NC + lax.axis_index("c")
    base = wid * PW
    pltpu.sync_copy(uid_hbm.at[pl.ds(base, PW)], uid_v)
    pltpu.sync_copy(iid_hbm.at[pl.ds(base, PW)], iid_v)

    taste_s = (taste0, taste1)
    attn_s = (attn0, attn1)
    item_s = (item0, item1)
    sems = (sem0, sem1)

    def fire(c, slot):
        cbase = c * C
        uidx = uid_v.at[pl.ds(cbase, C)]
        iidx = iid_v.at[pl.ds(cbase, C)]
        cps = [pltpu.async_copy(taste_hbm.at[uidx], taste_s[slot], sems[slot]),
               pltpu.async_copy(attn_hbm.at[uidx], attn_s[slot], sems[slot])]
        for j in range(D):
            cps.append(pltpu.async_copy(
                item_hbm.at[pl.ds(j * NIT, NIT)].at[iidx],
                item_s[slot].at[pl.ds(j * C, C)], sems[slot]))
        return cps

    def drain(slot):
        pltpu.make_async_copy(taste_hbm.at[pl.ds(0, C)], taste_s[slot],
                              sems[slot]).wait()
        pltpu.make_async_copy(attn_hbm.at[pl.ds(0, C)], attn_s[slot],
                              sems[slot]).wait()
        pltpu.make_async_copy(item_hbm.at[pl.ds(0, D * C)], item_s[slot],
                              sems[slot]).wait()

    cp_ub = pltpu.async_copy(ub_hbm.at[uid_v], ub_v, bsem)
    cp_ib = pltpu.async_copy(ib_hbm.at[iid_v], ib_v, bsem)
    fire(0, 0)
    fire(1, 1)
    cp_ub.wait()
    cp_ib.wait()

    def chunk_pair(it, carry):
        for slot in range(2):
            cbase = (it * 2 + slot) * C
            drain(slot)

            @pl.when(it < NCH // 2 - 1)
            def _(slot=slot):
                fire(it * 2 + slot + 2, slot)

            def group(g, carry2, cbase=cbase, slot=slot):
                taste_v, attn_v, item_b = (
                    taste_s[slot], attn_s[slot], item_s[slot])
                lane = lax.iota(jnp.int32, 16)
                rows = jnp.full((16,), g * 16, jnp.int32) + lane
                zero = jnp.zeros((16,), jnp.float32)
                logits = [zero, zero, zero, zero]
                tdots = [zero, zero, zero, zero]
                for j in range(D):
                    sk = (jnp.full((16,), j, jnp.int32) + lane) & (D - 1)
                    iv = plsc.load_gather(item_b, [sk * C + rows])
                    for m in range(M):
                        fc = sk + (m * D)
                        av = plsc.load_gather(attn_v, [rows, fc])
                        tv = plsc.load_gather(taste_v, [rows, fc])
                        logits[m] = logits[m] + av * iv
                        tdots[m] = tdots[m] + tv * iv
                mx = jnp.maximum(jnp.maximum(logits[0], logits[1]),
                                 jnp.maximum(logits[2], logits[3]))
                e = [jnp.exp(l - mx) for l in logits]
                num = (e[0] * tdots[0] + e[1] * tdots[1] + e[2] * tdots[2]
                       + e[3] * tdots[3])
                den = e[0] + e[1] + e[2] + e[3]
                obase = cbase + g * 16
                res = (num / den + ub_v[pl.ds(obase, 16)]
                       + ib_v[pl.ds(obase, 16)])
                out_v[pl.ds(obase, 16)] = res
                return carry2

            lax.fori_loop(0, GROUPS, group, None)
        return carry

    lax.fori_loop(0, NCH // 2, chunk_pair, None)

    pltpu.sync_copy(out_v, out_hbm.at[pl.ds(base, PW)])


_sc_call = pl.kernel(
    _body,
    out_type=jax.ShapeDtypeStruct((B,), jnp.float32),
    mesh=_mesh,
    scratch_types=[
        pltpu.VMEM((PW,), jnp.int32),
        pltpu.VMEM((PW,), jnp.int32),
        pltpu.VMEM((PW,), jnp.float32),
        pltpu.VMEM((PW,), jnp.float32),
        pltpu.VMEM((PW,), jnp.float32),
        pltpu.VMEM((C, M * D), jnp.float32),
        pltpu.VMEM((C, M * D), jnp.float32),
        pltpu.VMEM((C, M * D), jnp.float32),
        pltpu.VMEM((C, M * D), jnp.float32),
        pltpu.VMEM((D * C,), jnp.float32),
        pltpu.VMEM((D * C,), jnp.float32),
        pltpu.SemaphoreType.DMA,
        pltpu.SemaphoreType.DMA,
        pltpu.SemaphoreType.DMA,
    ],
    compiler_params=pltpu.CompilerParams(needs_layout_passes=False,
                                         use_tc_tiling_on_sc=False),
)


def kernel(user_ids, item_ids, taste_table, attention_table, item_table,
           user_biases, item_biases):
    uid = user_ids.astype(jnp.int32)
    iid = item_ids.astype(jnp.int32)
    ub = user_biases.reshape(-1)
    ib = item_biases.reshape(-1)
    item_feat = item_table.T.reshape(-1)
    return _sc_call(uid, iid, taste_table, attention_table, item_feat, ub, ib)
